# baseline (speedup 1.0000x reference)
import functools

import jax
import jax.numpy as jnp
from jax import lax
from jax.experimental import pallas as pl

SEQ = 200
EXPAND = 16
K_CH = 8
IG = 8


def _body(tsT_ref, tabh_ref, tabl_ref, out_ref):
    sT = tsT_ref[...]
    bsz = sT.shape[1]
    sb = lax.bitcast_convert_type(sT, jnp.int32)
    s = jnp.where(sb < 0, sb ^ jnp.int32(0x7FFFFFFF), sb)

    jio3 = lax.broadcasted_iota(jnp.int32, (K_CH, SEQ, 1), 1)
    kio3 = lax.broadcasted_iota(jnp.int32, (K_CH, SEQ, 1), 0)
    acc = jnp.zeros((SEQ, bsz), jnp.int32)
    for q in range(SEQ // K_CH):
        sk = lax.slice(s, (q * K_CH, 0), ((q + 1) * K_CH, bsz))
        m3 = (jio3 > kio3 + (q * K_CH)).astype(jnp.int32)
        cmp = (sk[:, None, :] - m3) < s[None]
        acc = acc + jnp.sum(cmp.astype(jnp.int32), axis=0)

    tabh = tabh_ref[...]
    tabl = tabl_ref[...]
    rt = jnp.concatenate([acc] * IG, axis=1)
    lane = lax.broadcasted_iota(jnp.int32, (1, IG * bsz), 1)
    dn = (((1,), (0,)), ((), ()))
    for ig in range(SEQ // IG):
        ii = lane // bsz + (ig * IG)
        mi = (rt == ii).astype(jnp.bfloat16)
        og = lax.dot_general(
            tabh, mi, dn, preferred_element_type=jnp.float32
        ) + lax.dot_general(
            tabl, mi, dn, preferred_element_type=jnp.float32
        )
        for t in range(IG):
            out_ref[ig * IG + t, :, :] = lax.slice(
                og, (0, t * bsz), (EXPAND, (t + 1) * bsz)
            )


@functools.partial(jax.jit, static_argnames=("block_b",))
def _run(tsT, tabT, block_b=512):
    batch = tsT.shape[1]
    grid = (batch // block_b,)
    tabh = tabT.astype(jnp.bfloat16)
    tabl = (tabT - tabh.astype(jnp.float32)).astype(jnp.bfloat16)
    return pl.pallas_call(
        _body,
        grid=grid,
        in_specs=[
            pl.BlockSpec((SEQ, block_b), lambda i: (0, i)),
            pl.BlockSpec((EXPAND, SEQ), lambda i: (0, 0)),
            pl.BlockSpec((EXPAND, SEQ), lambda i: (0, 0)),
        ],
        out_specs=pl.BlockSpec((SEQ, EXPAND, block_b), lambda i: (0, 0, i)),
        out_shape=jax.ShapeDtypeStruct((SEQ, EXPAND, batch), jnp.float32),
    )(tsT, tabh, tabl)


def kernel(ts, pos_embeddings):
    outT = _run(ts.T, pos_embeddings.T)
    return jnp.transpose(outT, (2, 0, 1))


# speedup vs baseline: 1.4009x; 1.4009x over previous
import functools

import jax
import jax.numpy as jnp
from jax import lax
from jax.experimental import pallas as pl

SEQ = 200
EXPAND = 16
K_CH = 8
IG = 8


def _body(tsT_ref, tabh_ref, tabl_ref, out_ref):
    sT = tsT_ref[...]
    bsz = sT.shape[1]
    sb = lax.bitcast_convert_type(sT, jnp.int32)
    s = jnp.where(sb < 0, sb ^ jnp.int32(0x7FFFFFFF), sb)

    jio3 = lax.broadcasted_iota(jnp.int32, (K_CH, SEQ, 1), 1)
    kio3 = lax.broadcasted_iota(jnp.int32, (K_CH, SEQ, 1), 0)
    acc = jnp.zeros((SEQ, bsz), jnp.int32)
    for q in range(SEQ // K_CH):
        sk = lax.slice(s, (q * K_CH, 0), ((q + 1) * K_CH, bsz))
        m3 = (jio3 > kio3 + (q * K_CH)).astype(jnp.int32)
        cmp = (sk[:, None, :] - m3) < s[None]
        acc = acc + jnp.sum(cmp.astype(jnp.int32), axis=0)

    tabh = tabh_ref[...]
    tabl = tabl_ref[...]
    rt = jnp.concatenate([acc] * IG, axis=1)
    lane = lax.broadcasted_iota(jnp.int32, (1, IG * bsz), 1)
    dn = (((1,), (0,)), ((), ()))
    for ig in range(SEQ // IG):
        ii = lane // bsz + (ig * IG)
        mi = (rt == ii).astype(jnp.bfloat16)
        og = lax.dot_general(
            tabh, mi, dn, preferred_element_type=jnp.float32
        )
        for t in range(IG):
            out_ref[ig * IG + t, :, :] = lax.slice(
                og, (0, t * bsz), (EXPAND, (t + 1) * bsz)
            )


@functools.partial(jax.jit, static_argnames=("block_b",))
def _run(tsT, tabT, block_b=512):
    batch = tsT.shape[1]
    grid = (batch // block_b,)
    tabh = tabT.astype(jnp.bfloat16)
    tabl = (tabT - tabh.astype(jnp.float32)).astype(jnp.bfloat16)
    return pl.pallas_call(
        _body,
        grid=grid,
        in_specs=[
            pl.BlockSpec((SEQ, block_b), lambda i: (0, i)),
            pl.BlockSpec((EXPAND, SEQ), lambda i: (0, 0)),
            pl.BlockSpec((EXPAND, SEQ), lambda i: (0, 0)),
        ],
        out_specs=pl.BlockSpec((SEQ, EXPAND, block_b), lambda i: (0, 0, i)),
        out_shape=jax.ShapeDtypeStruct((SEQ, EXPAND, batch), jnp.float32),
    )(tsT, tabh, tabl)


def kernel(ts, pos_embeddings):
    outT = _run(ts.T, pos_embeddings.T)
    return jnp.transpose(outT, (2, 0, 1))


# speedup vs baseline: 1.4207x; 1.0141x over previous
import functools

import jax
import jax.numpy as jnp
from jax import lax
from jax.experimental import pallas as pl

SEQ = 200
EXPAND = 16
K_CH = 8
IG = 8


def _body(tsT_ref, tabh_ref, out_ref):
    sT = tsT_ref[...]
    bsz = sT.shape[1]
    sb = lax.bitcast_convert_type(sT, jnp.int32)
    s = jnp.where(sb < 0, sb ^ jnp.int32(0x7FFFFFFF), sb)

    jio3 = lax.broadcasted_iota(jnp.int32, (K_CH, SEQ, 1), 1)
    kio3 = lax.broadcasted_iota(jnp.int32, (K_CH, SEQ, 1), 0)
    acc = jnp.zeros((SEQ, bsz), jnp.int32)
    for q in range(SEQ // K_CH):
        sk = lax.slice(s, (q * K_CH, 0), ((q + 1) * K_CH, bsz))
        m3 = (jio3 > kio3 + (q * K_CH)).astype(jnp.int32)
        cmp = (sk[:, None, :] - m3) < s[None]
        acc = acc + jnp.sum(cmp.astype(jnp.int32), axis=0)

    tabT = tabh_ref[...]
    rt = jnp.concatenate([acc] * IG, axis=1)
    lane = lax.broadcasted_iota(jnp.int32, (1, IG * bsz), 1)
    dn = (((1,), (0,)), ((), ()))
    for ig in range(SEQ // IG):
        ii = lane // bsz + (ig * IG)
        mi = (rt == ii).astype(jnp.float32)
        og = lax.dot_general(
            tabT, mi, dn, preferred_element_type=jnp.float32
        )
        for t in range(IG):
            out_ref[ig * IG + t, :, :] = lax.slice(
                og, (0, t * bsz), (EXPAND, (t + 1) * bsz)
            )


@functools.partial(jax.jit, static_argnames=("block_b",))
def _run(tsT, tabT, block_b=512):
    batch = tsT.shape[1]
    grid = (batch // block_b,)
    return pl.pallas_call(
        _body,
        grid=grid,
        in_specs=[
            pl.BlockSpec((SEQ, block_b), lambda i: (0, i)),
            pl.BlockSpec((EXPAND, SEQ), lambda i: (0, 0)),
        ],
        out_specs=pl.BlockSpec((SEQ, EXPAND, block_b), lambda i: (0, 0, i)),
        out_shape=jax.ShapeDtypeStruct((SEQ, EXPAND, batch), jnp.float32),
    )(tsT, tabT)


def kernel(ts, pos_embeddings):
    outT = _run(ts.T, pos_embeddings.T)
    return jnp.transpose(outT, (2, 0, 1))


# speedup vs baseline: 1.4209x; 1.0001x over previous
import functools

import jax
import jax.numpy as jnp
from jax import lax
from jax.experimental import pallas as pl

SEQ = 200
EXPAND = 16
K_CH = 8
IG = 8


def _body(tsT_ref, tabh_ref, out_ref):
    sT = tsT_ref[...]
    bsz = sT.shape[1]
    sb = lax.bitcast_convert_type(sT, jnp.int32)
    s = jnp.where(sb < 0, sb ^ jnp.int32(0x7FFFFFFF), sb)

    jio3 = lax.broadcasted_iota(jnp.int32, (K_CH, SEQ, 1), 1)
    kio3 = lax.broadcasted_iota(jnp.int32, (K_CH, SEQ, 1), 0)
    acc = jnp.zeros((SEQ, bsz), jnp.int32)
    for q in range(SEQ // K_CH):
        sk = lax.slice(s, (q * K_CH, 0), ((q + 1) * K_CH, bsz))
        m3 = (jio3 > kio3 + (q * K_CH)).astype(jnp.int32)
        cmp = (sk[:, None, :] - m3) < s[None]
        acc = acc + jnp.sum(cmp.astype(jnp.int32), axis=0)

    tabT = tabh_ref[...]
    rt = jnp.concatenate([acc] * IG, axis=1)
    lane = lax.broadcasted_iota(jnp.int32, (1, IG * bsz), 1)
    dn = (((1,), (0,)), ((), ()))
    for ig in range(SEQ // IG):
        ii = lane // bsz + (ig * IG)
        mi = (rt == ii).astype(jnp.float32)
        og = lax.dot_general(
            tabT, mi, dn, preferred_element_type=jnp.float32
        )
        for t in range(IG):
            out_ref[ig * IG + t, :, :] = lax.slice(
                og, (0, t * bsz), (EXPAND, (t + 1) * bsz)
            )


@functools.partial(jax.jit, static_argnames=("block_b",))
def _run(tsT, tabT, block_b=1024):
    batch = tsT.shape[1]
    grid = (batch // block_b,)
    return pl.pallas_call(
        _body,
        grid=grid,
        in_specs=[
            pl.BlockSpec((SEQ, block_b), lambda i: (0, i)),
            pl.BlockSpec((EXPAND, SEQ), lambda i: (0, 0)),
        ],
        out_specs=pl.BlockSpec((SEQ, EXPAND, block_b), lambda i: (0, 0, i)),
        out_shape=jax.ShapeDtypeStruct((SEQ, EXPAND, batch), jnp.float32),
    )(tsT, tabT)


def kernel(ts, pos_embeddings):
    outT = _run(ts.T, pos_embeddings.T)
    return jnp.transpose(outT, (2, 0, 1))
